# baseline (speedup 1.0000x reference)
import dataclasses
import functools

import jax
import jax.numpy as jnp
from jax.experimental import pallas as pl
from jax.experimental.pallas import tpu as pltpu
from jax.experimental.pallas import tpu_sc as plsc

_E = 8
_R = 16
_ALPHA = 2.0
_TB = 256
_LB = 1024
_NC = 2
_NS = 16
_L = 16
_W = _NC * _NS


def _logits_kernel(x_ref, wr_ref, out_ref):
    out_ref[...] = jnp.dot(x_ref[...], wr_ref[...],
                           preferred_element_type=jnp.float32)


def _sc_compiler_params():
    cp = pltpu.CompilerParams()
    if "needs_layout_passes" in pltpu.CompilerParams.__dataclass_fields__:
        cp = dataclasses.replace(cp, needs_layout_passes=False)
    return cp


def _make_route_sc(T):
    cpw = T // _W

    @functools.partial(
        pl.kernel,
        mesh=plsc.VectorSubcoreMesh(core_axis_name="c", subcore_axis_name="s"),
        out_type=jax.ShapeDtypeStruct((T * _E,), jnp.float32),
        scratch_types=[pltpu.VMEM((_E, cpw), jnp.float32),
                       pltpu.VMEM((cpw * _E,), jnp.float32)],
        compiler_params=_sc_compiler_params(),
    )
    def route(lt_hbm, out_hbm, lt, dw):
        wid = jax.lax.axis_index("s") * _NC + jax.lax.axis_index("c")
        base = wid * cpw
        pltpu.sync_copy(lt_hbm.at[:, pl.ds(base, cpw)], lt)

        @pl.loop(0, cpw * _E, step=_L)
        def _(i):
            dw[pl.ds(i, _L)] = jnp.zeros((_L,), jnp.float32)

        @pl.loop(0, cpw, step=_L)
        def _(t):
            vs = [lt[e, pl.ds(t, _L)] for e in range(_E)]
            m1 = vs[0]
            for e in range(1, _E):
                m1 = jnp.maximum(m1, vs[e])
            i1 = jnp.full((_L,), _E - 1, jnp.int32)
            for e in range(_E - 2, -1, -1):
                i1 = jnp.where(vs[e] == m1, e, i1)
            neg = jnp.full((_L,), -jnp.inf, jnp.float32)
            m2 = neg
            for e in range(_E):
                m2 = jnp.maximum(m2, jnp.where(i1 == e, neg, vs[e]))
            i2 = jnp.full((_L,), _E - 1, jnp.int32)
            for e in range(_E - 2, -1, -1):
                i2 = jnp.where((vs[e] == m2) & (i1 != e), e, i2)
            w1 = 1.0 / (1.0 + jnp.exp(m2 - m1))
            w2 = 1.0 - w1
            tok = jax.lax.iota(jnp.int32, _L) + t
            plsc.store_scatter(dw, [tok * _E + i1], w1)
            plsc.store_scatter(dw, [tok * _E + i2], w2)

        pltpu.sync_copy(dw, out_hbm.at[pl.ds(base * _E, cpw * _E)])

    return route


def _moe_lora_kernel(x_ref, dw_ref, wup_ref, wgate_ref, wdown_ref,
                     b1c_ref, a1_ref, b3c_ref, a3_ref, b2_ref, a2c_ref,
                     out_ref):
    f32 = jnp.float32
    bf16 = jnp.bfloat16
    xb = x_ref[...]
    dwb = dw_ref[...].astype(bf16)

    U = jnp.dot(xb, wup_ref[...], preferred_element_type=f32)
    G = jnp.dot(xb, wgate_ref[...], preferred_element_type=f32)
    XB1 = jnp.dot(xb, b1c_ref[...], preferred_element_type=f32)
    XB3 = jnp.dot(xb, b3c_ref[...], preferred_element_type=f32)
    XB1 = XB1.astype(bf16)
    XB3 = XB3.astype(bf16)

    H = jnp.zeros(U.shape, f32)
    qs = []
    for e in range(_E):
        p1 = jnp.dot(XB1[:, e * _R:(e + 1) * _R], a1_ref[e],
                     preferred_element_type=f32)
        p3 = jnp.dot(XB3[:, e * _R:(e + 1) * _R], a3_ref[e],
                     preferred_element_type=f32)
        z = (U + p1).astype(bf16)
        g = (G + p3).astype(bf16)
        h = (z * jax.nn.sigmoid(z)) * g
        hw = h * dwb[:, e:e + 1]
        H = H + hw.astype(f32)
        qs.append(jnp.dot(hw, b2_ref[e], preferred_element_type=f32))
    Q = jnp.concatenate(qs, axis=1)
    out = jnp.dot(H.astype(bf16), wdown_ref[...], preferred_element_type=f32)
    out = out + jnp.dot(Q.astype(bf16), a2c_ref[...],
                        preferred_element_type=f32)
    out_ref[...] = out


def kernel(x, W_up, W_gate_proj, W_down, W_router, A1, B1, A2, B2, A3, B3):
    T, D = x.shape
    F = W_up.shape[0]
    bf16 = jnp.bfloat16
    xb = x.astype(bf16)
    wr = W_router.T.astype(bf16)
    wup = W_up.T.astype(bf16)
    wgate = W_gate_proj.T.astype(bf16)
    wdown = W_down.T.astype(bf16)
    b1c = B1.transpose(2, 0, 1).reshape(F, _E * _R).astype(bf16)
    b3c = B3.transpose(2, 0, 1).reshape(D, _E * _R).astype(bf16)
    a1t = (_ALPHA * A1.transpose(0, 2, 1)).astype(bf16)
    a3t = (_ALPHA * A3.transpose(0, 2, 1)).astype(bf16)
    b2t = B2.transpose(0, 2, 1).astype(bf16)
    a2c = (_ALPHA * A2.transpose(0, 2, 1)).reshape(_E * _R, D).astype(bf16)

    logits = pl.pallas_call(
        _logits_kernel,
        grid=(T // _LB,),
        in_specs=[
            pl.BlockSpec((_LB, D), lambda i: (i, 0)),
            pl.BlockSpec((D, _E), lambda i: (0, 0)),
        ],
        out_specs=pl.BlockSpec((_LB, _E), lambda i: (i, 0)),
        out_shape=jax.ShapeDtypeStruct((T, _E), jnp.float32),
    )(xb, wr)

    dw = _make_route_sc(T)(logits.T).reshape(T, _E)

    out = pl.pallas_call(
        _moe_lora_kernel,
        grid=(T // _TB,),
        in_specs=[
            pl.BlockSpec((_TB, D), lambda i: (i, 0)),
            pl.BlockSpec((_TB, _E), lambda i: (i, 0)),
            pl.BlockSpec((D, F), lambda i: (0, 0)),
            pl.BlockSpec((D, F), lambda i: (0, 0)),
            pl.BlockSpec((F, D), lambda i: (0, 0)),
            pl.BlockSpec((D, _E * _R), lambda i: (0, 0)),
            pl.BlockSpec((_E, _R, F), lambda i: (0, 0, 0)),
            pl.BlockSpec((D, _E * _R), lambda i: (0, 0)),
            pl.BlockSpec((_E, _R, F), lambda i: (0, 0, 0)),
            pl.BlockSpec((_E, F, _R), lambda i: (0, 0, 0)),
            pl.BlockSpec((_E * _R, D), lambda i: (0, 0)),
        ],
        out_specs=pl.BlockSpec((_TB, D), lambda i: (i, 0)),
        out_shape=jax.ShapeDtypeStruct((T, D), jnp.float32),
    )(xb, dw, wup, wgate, wdown, b1c, a1t, b3c, a3t, b2t, a2c)
    return out


# speedup vs baseline: 1.0066x; 1.0066x over previous
import dataclasses
import functools

import jax
import jax.numpy as jnp
from jax.experimental import pallas as pl
from jax.experimental.pallas import tpu as pltpu
from jax.experimental.pallas import tpu_sc as plsc

_E = 8
_R = 16
_ALPHA = 2.0
_TB = 256
_LB = 1024
_NC = 2
_NS = 16
_L = 16
_W = _NC * _NS


def _logits_kernel(x_ref, wr_ref, out_ref):
    l = jnp.dot(x_ref[...], wr_ref[...], preferred_element_type=jnp.float32)
    out_ref[...] = l.T


def _sc_compiler_params():
    cp = pltpu.CompilerParams()
    if "needs_layout_passes" in pltpu.CompilerParams.__dataclass_fields__:
        cp = dataclasses.replace(cp, needs_layout_passes=False)
    return cp


def _make_route_sc(T):
    cpw = T // _W

    @functools.partial(
        pl.kernel,
        mesh=plsc.VectorSubcoreMesh(core_axis_name="c", subcore_axis_name="s"),
        out_type=jax.ShapeDtypeStruct((T * _E,), jnp.float32),
        scratch_types=[pltpu.VMEM((_E, cpw), jnp.float32),
                       pltpu.VMEM((cpw * _E,), jnp.float32)],
        compiler_params=_sc_compiler_params(),
    )
    def route(lt_hbm, out_hbm, lt, dw):
        wid = jax.lax.axis_index("s") * _NC + jax.lax.axis_index("c")
        base = wid * cpw
        pltpu.sync_copy(lt_hbm.at[:, pl.ds(base, cpw)], lt)

        @pl.loop(0, cpw * _E, step=_L)
        def _(i):
            dw[pl.ds(i, _L)] = jnp.zeros((_L,), jnp.float32)

        @pl.loop(0, cpw, step=_L)
        def _(t):
            vs = [lt[e, pl.ds(t, _L)] for e in range(_E)]
            m1 = vs[0]
            for e in range(1, _E):
                m1 = jnp.maximum(m1, vs[e])
            i1 = jnp.full((_L,), _E - 1, jnp.int32)
            for e in range(_E - 2, -1, -1):
                i1 = jnp.where(vs[e] == m1, e, i1)
            neg = jnp.full((_L,), -jnp.inf, jnp.float32)
            m2 = neg
            for e in range(_E):
                m2 = jnp.maximum(m2, jnp.where(i1 == e, neg, vs[e]))
            i2 = jnp.full((_L,), _E - 1, jnp.int32)
            for e in range(_E - 2, -1, -1):
                i2 = jnp.where((vs[e] == m2) & (i1 != e), e, i2)
            w1 = 1.0 / (1.0 + jnp.exp(m2 - m1))
            w2 = 1.0 - w1
            tok = jax.lax.iota(jnp.int32, _L) + t
            plsc.store_scatter(dw, [tok * _E + i1], w1)
            plsc.store_scatter(dw, [tok * _E + i2], w2)

        pltpu.sync_copy(dw, out_hbm.at[pl.ds(base * _E, cpw * _E)])

    return route


def _moe_lora_kernel(x_ref, dw_ref, wup_ref, wgate_ref, wdown_ref,
                     b1c_ref, a1_ref, b3c_ref, a3_ref, b2_ref, a2c_ref,
                     out_ref):
    f32 = jnp.float32
    bf16 = jnp.bfloat16
    xb = x_ref[...]
    dwb = dw_ref[...].astype(bf16)

    U = jnp.dot(xb, wup_ref[...], preferred_element_type=f32)
    G = jnp.dot(xb, wgate_ref[...], preferred_element_type=f32)
    XB1 = jnp.dot(xb, b1c_ref[...], preferred_element_type=f32)
    XB3 = jnp.dot(xb, b3c_ref[...], preferred_element_type=f32)
    XB1 = XB1.astype(bf16)
    XB3 = XB3.astype(bf16)

    H = jnp.zeros(U.shape, f32)
    qs = []
    for e in range(_E):
        p1 = jnp.dot(XB1[:, e * _R:(e + 1) * _R], a1_ref[e],
                     preferred_element_type=f32)
        p3 = jnp.dot(XB3[:, e * _R:(e + 1) * _R], a3_ref[e],
                     preferred_element_type=f32)
        z = (U + p1).astype(bf16)
        g = (G + p3).astype(bf16)
        h = (z * jax.nn.sigmoid(z)) * g
        hw = h * dwb[:, e:e + 1]
        H = H + hw.astype(f32)
        qs.append(jnp.dot(hw, b2_ref[e], preferred_element_type=f32))
    Q = jnp.concatenate(qs, axis=1)
    out = jnp.dot(H.astype(bf16), wdown_ref[...], preferred_element_type=f32)
    out = out + jnp.dot(Q.astype(bf16), a2c_ref[...],
                        preferred_element_type=f32)
    out_ref[...] = out


def kernel(x, W_up, W_gate_proj, W_down, W_router, A1, B1, A2, B2, A3, B3):
    T, D = x.shape
    F = W_up.shape[0]
    bf16 = jnp.bfloat16
    xb = x.astype(bf16)
    wr = W_router.T.astype(bf16)
    wup = W_up.T.astype(bf16)
    wgate = W_gate_proj.T.astype(bf16)
    wdown = W_down.T.astype(bf16)
    b1c = B1.transpose(2, 0, 1).reshape(F, _E * _R).astype(bf16)
    b3c = B3.transpose(2, 0, 1).reshape(D, _E * _R).astype(bf16)
    a1t = (_ALPHA * A1.transpose(0, 2, 1)).astype(bf16)
    a3t = (_ALPHA * A3.transpose(0, 2, 1)).astype(bf16)
    b2t = B2.transpose(0, 2, 1).astype(bf16)
    a2c = (_ALPHA * A2.transpose(0, 2, 1)).reshape(_E * _R, D).astype(bf16)

    logits_t = pl.pallas_call(
        _logits_kernel,
        grid=(T // _LB,),
        in_specs=[
            pl.BlockSpec((_LB, D), lambda i: (i, 0)),
            pl.BlockSpec((D, _E), lambda i: (0, 0)),
        ],
        out_specs=pl.BlockSpec((_E, _LB), lambda i: (0, i)),
        out_shape=jax.ShapeDtypeStruct((_E, T), jnp.float32),
    )(xb, wr)

    dw = _make_route_sc(T)(logits_t).reshape(T, _E)

    out = pl.pallas_call(
        _moe_lora_kernel,
        grid=(T // _TB,),
        in_specs=[
            pl.BlockSpec((_TB, D), lambda i: (i, 0)),
            pl.BlockSpec((_TB, _E), lambda i: (i, 0)),
            pl.BlockSpec((D, F), lambda i: (0, 0)),
            pl.BlockSpec((D, F), lambda i: (0, 0)),
            pl.BlockSpec((F, D), lambda i: (0, 0)),
            pl.BlockSpec((D, _E * _R), lambda i: (0, 0)),
            pl.BlockSpec((_E, _R, F), lambda i: (0, 0, 0)),
            pl.BlockSpec((D, _E * _R), lambda i: (0, 0)),
            pl.BlockSpec((_E, _R, F), lambda i: (0, 0, 0)),
            pl.BlockSpec((_E, F, _R), lambda i: (0, 0, 0)),
            pl.BlockSpec((_E * _R, D), lambda i: (0, 0)),
        ],
        out_specs=pl.BlockSpec((_TB, D), lambda i: (i, 0)),
        out_shape=jax.ShapeDtypeStruct((T, D), jnp.float32),
    )(xb, dw, wup, wgate, wdown, b1c, a1t, b3c, a3t, b2t, a2c)
    return out


# speedup vs baseline: 1.0089x; 1.0023x over previous
import dataclasses
import functools

import jax
import jax.numpy as jnp
from jax.experimental import pallas as pl
from jax.experimental.pallas import tpu as pltpu
from jax.experimental.pallas import tpu_sc as plsc

_E = 8
_R = 16
_ALPHA = 2.0
_TB = 256
_LB = 1024
_NC = 2
_NS = 16
_L = 16
_W = _NC * _NS


def _logits_kernel(x_ref, wr_ref, out_ref):
    l = jnp.dot(x_ref[...], wr_ref[...], preferred_element_type=jnp.float32)
    out_ref[...] = l.T


def _sc_compiler_params():
    cp = pltpu.CompilerParams()
    if "needs_layout_passes" in pltpu.CompilerParams.__dataclass_fields__:
        cp = dataclasses.replace(cp, needs_layout_passes=False)
    return cp


def _make_route_sc(T):
    cpw = T // _W

    @functools.partial(
        pl.kernel,
        mesh=plsc.VectorSubcoreMesh(core_axis_name="c", subcore_axis_name="s"),
        out_type=jax.ShapeDtypeStruct((T * _E,), jnp.float32),
        scratch_types=[pltpu.VMEM((_E, cpw), jnp.float32),
                       pltpu.VMEM((cpw * _E,), jnp.float32)],
        compiler_params=_sc_compiler_params(),
    )
    def route(lt_hbm, out_hbm, lt, dw):
        wid = jax.lax.axis_index("s") * _NC + jax.lax.axis_index("c")
        base = wid * cpw
        pltpu.sync_copy(lt_hbm.at[:, pl.ds(base, cpw)], lt)

        @pl.loop(0, cpw * _E, step=_L)
        def _(i):
            dw[pl.ds(i, _L)] = jnp.zeros((_L,), jnp.float32)

        @pl.loop(0, cpw, step=_L)
        def _(t):
            vs = [lt[e, pl.ds(t, _L)] for e in range(_E)]
            m1 = vs[0]
            for e in range(1, _E):
                m1 = jnp.maximum(m1, vs[e])
            i1 = jnp.full((_L,), _E - 1, jnp.int32)
            for e in range(_E - 2, -1, -1):
                i1 = jnp.where(vs[e] == m1, e, i1)
            neg = jnp.full((_L,), -jnp.inf, jnp.float32)
            m2 = neg
            for e in range(_E):
                m2 = jnp.maximum(m2, jnp.where(i1 == e, neg, vs[e]))
            i2 = jnp.full((_L,), _E - 1, jnp.int32)
            for e in range(_E - 2, -1, -1):
                i2 = jnp.where((vs[e] == m2) & (i1 != e), e, i2)
            w1 = 1.0 / (1.0 + jnp.exp(m2 - m1))
            w2 = 1.0 - w1
            tok = jax.lax.iota(jnp.int32, _L) + t
            plsc.store_scatter(dw, [tok * _E + i1], w1)
            plsc.store_scatter(dw, [tok * _E + i2], w2)

        pltpu.sync_copy(dw, out_hbm.at[pl.ds(base * _E, cpw * _E)])

    return route


def _moe_lora_kernel(x_ref, dw_ref, wup_ref, wgate_ref, wdown_ref,
                     b1c_ref, a1_ref, b3c_ref, a3_ref, b2_ref, a2c_ref,
                     out_ref):
    f32 = jnp.float32
    bf16 = jnp.bfloat16
    xb = x_ref[...]
    dwb = dw_ref[...].astype(bf16)

    U = jnp.dot(xb, wup_ref[...], preferred_element_type=f32)
    G = jnp.dot(xb, wgate_ref[...], preferred_element_type=f32)
    XB1 = jnp.dot(xb, b1c_ref[...], preferred_element_type=f32)
    XB3 = jnp.dot(xb, b3c_ref[...], preferred_element_type=f32)
    XB1 = XB1.astype(bf16)
    XB3 = XB3.astype(bf16)
    Ub = U.astype(bf16)
    Gb = G.astype(bf16)

    H = jnp.zeros(U.shape, f32)
    qs = []
    for e in range(_E):
        p1 = jnp.dot(XB1[:, e * _R:(e + 1) * _R], a1_ref[e],
                     preferred_element_type=f32)
        p3 = jnp.dot(XB3[:, e * _R:(e + 1) * _R], a3_ref[e],
                     preferred_element_type=f32)
        z = Ub + p1.astype(bf16)
        g = Gb + p3.astype(bf16)
        h = (z * jax.nn.sigmoid(z)) * g
        hw = h * dwb[:, e:e + 1]
        H = H + hw.astype(f32)
        qs.append(jnp.dot(hw, b2_ref[e], preferred_element_type=f32))
    Q = jnp.concatenate(qs, axis=1)
    out = jnp.dot(H.astype(bf16), wdown_ref[...], preferred_element_type=f32)
    out = out + jnp.dot(Q.astype(bf16), a2c_ref[...],
                        preferred_element_type=f32)
    out_ref[...] = out


def kernel(x, W_up, W_gate_proj, W_down, W_router, A1, B1, A2, B2, A3, B3):
    T, D = x.shape
    F = W_up.shape[0]
    bf16 = jnp.bfloat16
    xb = x.astype(bf16)
    wr = W_router.T.astype(bf16)
    wup = W_up.T.astype(bf16)
    wgate = W_gate_proj.T.astype(bf16)
    wdown = W_down.T.astype(bf16)
    b1c = B1.transpose(2, 0, 1).reshape(F, _E * _R).astype(bf16)
    b3c = B3.transpose(2, 0, 1).reshape(D, _E * _R).astype(bf16)
    a1t = (_ALPHA * A1.transpose(0, 2, 1)).astype(bf16)
    a3t = (_ALPHA * A3.transpose(0, 2, 1)).astype(bf16)
    b2t = B2.transpose(0, 2, 1).astype(bf16)
    a2c = (_ALPHA * A2.transpose(0, 2, 1)).reshape(_E * _R, D).astype(bf16)

    logits_t = pl.pallas_call(
        _logits_kernel,
        grid=(T // _LB,),
        in_specs=[
            pl.BlockSpec((_LB, D), lambda i: (i, 0)),
            pl.BlockSpec((D, _E), lambda i: (0, 0)),
        ],
        out_specs=pl.BlockSpec((_E, _LB), lambda i: (0, i)),
        out_shape=jax.ShapeDtypeStruct((_E, T), jnp.float32),
    )(xb, wr)

    dw = _make_route_sc(T)(logits_t).reshape(T, _E)

    out = pl.pallas_call(
        _moe_lora_kernel,
        grid=(T // _TB,),
        in_specs=[
            pl.BlockSpec((_TB, D), lambda i: (i, 0)),
            pl.BlockSpec((_TB, _E), lambda i: (i, 0)),
            pl.BlockSpec((D, F), lambda i: (0, 0)),
            pl.BlockSpec((D, F), lambda i: (0, 0)),
            pl.BlockSpec((F, D), lambda i: (0, 0)),
            pl.BlockSpec((D, _E * _R), lambda i: (0, 0)),
            pl.BlockSpec((_E, _R, F), lambda i: (0, 0, 0)),
            pl.BlockSpec((D, _E * _R), lambda i: (0, 0)),
            pl.BlockSpec((_E, _R, F), lambda i: (0, 0, 0)),
            pl.BlockSpec((_E, F, _R), lambda i: (0, 0, 0)),
            pl.BlockSpec((_E * _R, D), lambda i: (0, 0)),
        ],
        out_specs=pl.BlockSpec((_TB, D), lambda i: (i, 0)),
        out_shape=jax.ShapeDtypeStruct((T, D), jnp.float32),
    )(xb, dw, wup, wgate, wdown, b1c, a1t, b3c, a3t, b2t, a2c)
    return out
